# Initial kernel scaffold; baseline (speedup 1.0000x reference)
import functools

import jax
import jax.numpy as jnp
from jax import lax
from jax.experimental import pallas as pl
from jax.experimental.pallas import tpu as pltpu
from jax.experimental.pallas import tpu_sc as plsc

N = 10000
E = 320000
D = 128

NC = 2
NS = 16
L = 16
NW = NC * NS

NP = 10240
EW = E // NW
K = 80
NCHUNK = EW // K
ROWS_PER_TILE = NP // NS

_MESH = plsc.VectorSubcoreMesh(
    core_axis_name="c", subcore_axis_name="s", num_cores=NC, num_subcores=NS
)


def _rsqrt16(d):
    i = lax.bitcast_convert_type(d, jnp.int32)
    i = jnp.int32(0x5F3759DF) - lax.shift_right_logical(i, 1)
    r = lax.bitcast_convert_type(i, jnp.float32)
    half = d * jnp.float32(0.5)
    for _ in range(3):
        r = r * (jnp.float32(1.5) - half * r * r)
    return jnp.where(d == jnp.float32(0.0), jnp.float32(jnp.inf), r)


def _degree_body(n1_hbm, degp_hbm, idx_v, ones_v, zero_v, shared_deg, sem):
    c = lax.axis_index("c")
    s = lax.axis_index("s")
    wid = s * NC + c
    base = wid * EW

    for k in range(K // L):
        ones_v[pl.ds(k * L, L)] = jnp.full((L,), 1.0, jnp.float32)
        zero_v[pl.ds(k * L, L)] = jnp.full((L,), 0.0, jnp.float32)

    for k in range(ROWS_PER_TILE // K):
        pltpu.sync_copy(zero_v, shared_deg.at[pl.ds(s * ROWS_PER_TILE + k * K, K)])
    plsc.subcore_barrier()

    def chunk(i, carry):
        pltpu.sync_copy(n1_hbm.at[pl.ds(base + i * K, K)], idx_v)
        pltpu.async_copy(ones_v, shared_deg.at[idx_v], sem, add=True).wait()
        return carry

    lax.fori_loop(0, NCHUNK, chunk, 0)
    plsc.subcore_barrier()

    pltpu.sync_copy(
        shared_deg.at[pl.ds(s * ROWS_PER_TILE, ROWS_PER_TILE)],
        degp_hbm.at[c, pl.ds(s * ROWS_PER_TILE, ROWS_PER_TILE)],
    )


_degree_kernel = pl.kernel(
    _degree_body,
    out_type=jax.ShapeDtypeStruct((NC, NP), jnp.float32),
    mesh=_MESH,
    scratch_types=[
        pltpu.VMEM((K,), jnp.int32),
        pltpu.VMEM((K,), jnp.float32),
        pltpu.VMEM((K,), jnp.float32),
        pltpu.VMEM_SHARED((NP,), jnp.float32),
        pltpu.SemaphoreType.DMA,
    ],
)


def _edge_body(
    poi_hbm, n1_hbm, n2_hbm, dist_hbm, degp_hbm, xpart_hbm,
    invdeg_v, tmp_v, idx1_v, idx2_v, dist_v, w_v, rows_v, shared_acc, sem,
):
    c = lax.axis_index("c")
    s = lax.axis_index("s")
    wid = s * NC + c
    base = wid * EW

    pltpu.sync_copy(degp_hbm.at[0], invdeg_v)
    pltpu.sync_copy(degp_hbm.at[1], tmp_v)

    def invdeg_step(i, carry):
        d = invdeg_v[pl.ds(i * L, L)] + tmp_v[pl.ds(i * L, L)]
        invdeg_v[pl.ds(i * L, L)] = _rsqrt16(d)
        return carry

    lax.fori_loop(0, NP // L, invdeg_step, 0)

    for e in range(K):
        for j in range(D // L):
            rows_v[e, pl.ds(j * L, L)] = jnp.full((L,), 0.0, jnp.float32)
    for k in range(ROWS_PER_TILE // K):
        pltpu.sync_copy(rows_v, shared_acc.at[pl.ds(s * ROWS_PER_TILE + k * K, K)])
    plsc.subcore_barrier()

    def chunk(i, carry):
        off = base + i * K
        pltpu.sync_copy(n1_hbm.at[pl.ds(off, K)], idx1_v)
        pltpu.sync_copy(n2_hbm.at[pl.ds(off, K)], idx2_v)
        pltpu.sync_copy(dist_hbm.at[pl.ds(off, K)], dist_v)
        pltpu.async_copy(poi_hbm.at[idx2_v], rows_v, sem).wait()
        for k in range(K // L):
            n1v = idx1_v[pl.ds(k * L, L)]
            n2v = idx2_v[pl.ds(k * L, L)]
            w1 = plsc.load_gather(invdeg_v, [n1v])
            w2 = plsc.load_gather(invdeg_v, [n2v])
            d = dist_v[pl.ds(k * L, L)]
            w_v[pl.ds(k * L, L)] = w1 * w2 * jnp.exp(-d * d)
        for e in range(K):
            ws = plsc.load_gather(w_v, [jnp.full((L,), e, jnp.int32)])
            for j in range(D // L):
                rows_v[e, pl.ds(j * L, L)] = rows_v[e, pl.ds(j * L, L)] * ws
        pltpu.async_copy(rows_v, shared_acc.at[idx1_v], sem, add=True).wait()
        return carry

    lax.fori_loop(0, NCHUNK, chunk, 0)
    plsc.subcore_barrier()

    pltpu.sync_copy(
        shared_acc.at[pl.ds(s * ROWS_PER_TILE, ROWS_PER_TILE)],
        xpart_hbm.at[c, pl.ds(s * ROWS_PER_TILE, ROWS_PER_TILE)],
    )


_edge_kernel = pl.kernel(
    _edge_body,
    out_type=jax.ShapeDtypeStruct((NC, NP, D), jnp.float32),
    mesh=_MESH,
    scratch_types=[
        pltpu.VMEM((NP,), jnp.float32),
        pltpu.VMEM((NP,), jnp.float32),
        pltpu.VMEM((K,), jnp.int32),
        pltpu.VMEM((K,), jnp.int32),
        pltpu.VMEM((K,), jnp.float32),
        pltpu.VMEM((K,), jnp.float32),
        pltpu.VMEM((K, D), jnp.float32),
        pltpu.VMEM_SHARED((NP, D), jnp.float32),
        pltpu.SemaphoreType.DMA,
    ],
)


_BM = 1024


def _matmul_body(x_ref, w_ref, b_ref, o_ref):
    xs = x_ref[0] + x_ref[1]
    acc = lax.dot_general(
        xs, w_ref[...], (((1,), (1,)), ((), ())),
        preferred_element_type=jnp.float32,
    )
    o_ref[...] = acc + b_ref[...]


def _matmul(xpart, W, b2d):
    return pl.pallas_call(
        _matmul_body,
        grid=(NP // _BM,),
        in_specs=[
            pl.BlockSpec((NC, _BM, D), lambda i: (0, i, 0)),
            pl.BlockSpec((D, D), lambda i: (0, 0)),
            pl.BlockSpec((1, D), lambda i: (0, 0)),
        ],
        out_specs=pl.BlockSpec((_BM, D), lambda i: (i, 0)),
        out_shape=jax.ShapeDtypeStruct((NP, D), jnp.float32),
    )(xpart, W, b2d)


@jax.jit
def kernel(poi_rep, edge_index, dist_vec, W, b):
    n1 = edge_index[0]
    n2 = edge_index[1]
    degp = _degree_kernel(n1)
    xpart = _edge_kernel(poi_rep, n1, n2, dist_vec, degp)
    msg = _matmul(xpart, W, jnp.reshape(b, (1, D)))
    return msg[:N]


# speedup vs baseline: 11.6637x; 11.6637x over previous
import functools

import jax
import jax.numpy as jnp
from jax import lax
from jax.experimental import pallas as pl
from jax.experimental.pallas import tpu as pltpu
from jax.experimental.pallas import tpu_sc as plsc

N = 10000
E = 320000
D = 128

NC = 2
NS = 16
L = 16
NW = NC * NS

NP = 10240
EW = E // NW
K = 80
NCHUNK = EW // K
ROWS_PER_TILE = NP // NS

_MESH = plsc.VectorSubcoreMesh(
    core_axis_name="c", subcore_axis_name="s", num_cores=NC, num_subcores=NS
)


def _rsqrt16(d):
    i = lax.bitcast_convert_type(d, jnp.int32)
    i = jnp.int32(0x5F3759DF) - lax.shift_right_logical(i, 1)
    r = lax.bitcast_convert_type(i, jnp.float32)
    half = d * jnp.float32(0.5)
    for _ in range(3):
        r = r * (jnp.float32(1.5) - half * r * r)
    return jnp.where(d == jnp.float32(0.0), jnp.float32(jnp.inf), r)


def _degree_body(n1_hbm, degp_hbm, idx_v, ones_v, zero_v, shared_deg, sem):
    c = lax.axis_index("c")
    s = lax.axis_index("s")
    wid = s * NC + c
    base = wid * EW

    for k in range(K // L):
        ones_v[pl.ds(k * L, L)] = jnp.full((L,), 1.0, jnp.float32)
        zero_v[pl.ds(k * L, L)] = jnp.full((L,), 0.0, jnp.float32)

    for k in range(ROWS_PER_TILE // K):
        pltpu.sync_copy(zero_v, shared_deg.at[pl.ds(s * ROWS_PER_TILE + k * K, K)])
    plsc.subcore_barrier()

    def chunk(i, carry):
        pltpu.sync_copy(n1_hbm.at[pl.ds(base + i * K, K)], idx_v)
        pltpu.async_copy(ones_v, shared_deg.at[idx_v], sem, add=True).wait()
        return carry

    lax.fori_loop(0, NCHUNK, chunk, 0)
    plsc.subcore_barrier()

    pltpu.sync_copy(
        shared_deg.at[pl.ds(s * ROWS_PER_TILE, ROWS_PER_TILE)],
        degp_hbm.at[c, pl.ds(s * ROWS_PER_TILE, ROWS_PER_TILE)],
    )


_SC_PARAMS = pltpu.CompilerParams(needs_layout_passes=False)

_degree_kernel = pl.kernel(
    _degree_body,
    out_type=jax.ShapeDtypeStruct((NC, NP), jnp.float32),
    mesh=_MESH,
    compiler_params=_SC_PARAMS,
    scratch_types=[
        pltpu.VMEM((K,), jnp.int32),
        pltpu.VMEM((K,), jnp.float32),
        pltpu.VMEM((K,), jnp.float32),
        pltpu.VMEM_SHARED((NP,), jnp.float32),
        pltpu.SemaphoreType.DMA,
    ],
)


def _edge_body(
    poi_hbm, n1_hbm, n2_hbm, dist_hbm, degp_hbm, xpart_hbm,
    invdeg_v, tmp_v, idx1_v, idx2_v, dist_v, rows_v, shared_acc,
    sem_g, sem_s,
):
    c = lax.axis_index("c")
    s = lax.axis_index("s")
    wid = s * NC + c
    base = wid * EW

    pltpu.sync_copy(degp_hbm.at[0], invdeg_v)
    pltpu.sync_copy(degp_hbm.at[1], tmp_v)

    def invdeg_step(i, carry):
        d = invdeg_v[pl.ds(i * L, L)] + tmp_v[pl.ds(i * L, L)]
        invdeg_v[pl.ds(i * L, L)] = _rsqrt16(d)
        return carry

    lax.fori_loop(0, NP // L, invdeg_step, 0)

    for e in range(K):
        for j in range(D // L):
            rows_v[e, pl.ds(j * L, L)] = jnp.full((L,), 0.0, jnp.float32)
    for k in range(ROWS_PER_TILE // K):
        pltpu.sync_copy(rows_v, shared_acc.at[pl.ds(s * ROWS_PER_TILE + k * K, K)])
    plsc.subcore_barrier()

    def chunk(i, carry):
        off = base + i * K
        pltpu.sync_copy(n1_hbm.at[pl.ds(off, K)], idx1_v)
        pltpu.sync_copy(n2_hbm.at[pl.ds(off, K)], idx2_v)
        pltpu.sync_copy(dist_hbm.at[pl.ds(off, K)], dist_v)
        pltpu.async_copy(poi_hbm.at[idx2_v], rows_v, sem_g).wait()
        for k in range(K // L):
            n1v = idx1_v[pl.ds(k * L, L)]
            n2v = idx2_v[pl.ds(k * L, L)]
            w1 = plsc.load_gather(invdeg_v, [n1v])
            w2 = plsc.load_gather(invdeg_v, [n2v])
            d = dist_v[pl.ds(k * L, L)]
            wv = w1 * w2 * jnp.exp(-d * d)
            for l in range(L):
                e = k * L + l
                ws = jnp.full((L,), wv[l], jnp.float32)
                for j in range(D // L):
                    rows_v[e, pl.ds(j * L, L)] = rows_v[e, pl.ds(j * L, L)] * ws
        pltpu.async_copy(rows_v, shared_acc.at[idx1_v], sem_s, add=True).wait()
        return carry

    lax.fori_loop(0, NCHUNK, chunk, 0)
    plsc.subcore_barrier()

    pltpu.sync_copy(
        shared_acc.at[pl.ds(s * ROWS_PER_TILE, ROWS_PER_TILE)],
        xpart_hbm.at[c, pl.ds(s * ROWS_PER_TILE, ROWS_PER_TILE)],
    )


_edge_kernel = pl.kernel(
    _edge_body,
    out_type=jax.ShapeDtypeStruct((NC, NP, D), jnp.float32),
    mesh=_MESH,
    compiler_params=_SC_PARAMS,
    scratch_types=[
        pltpu.VMEM((NP,), jnp.float32),
        pltpu.VMEM((NP,), jnp.float32),
        pltpu.VMEM((K,), jnp.int32),
        pltpu.VMEM((K,), jnp.int32),
        pltpu.VMEM((K,), jnp.float32),
        pltpu.VMEM((K, D), jnp.float32),
        pltpu.VMEM_SHARED((NP, D), jnp.float32),
        pltpu.SemaphoreType.DMA,
        pltpu.SemaphoreType.DMA,
    ],
)


_BM = 1024


def _matmul_body(x_ref, w_ref, b_ref, o_ref):
    xs = x_ref[0] + x_ref[1]
    acc = lax.dot_general(
        xs, w_ref[...], (((1,), (1,)), ((), ())),
        preferred_element_type=jnp.float32,
    )
    o_ref[...] = acc + b_ref[...]


def _matmul(xpart, W, b2d):
    return pl.pallas_call(
        _matmul_body,
        grid=(NP // _BM,),
        in_specs=[
            pl.BlockSpec((NC, _BM, D), lambda i: (0, i, 0)),
            pl.BlockSpec((D, D), lambda i: (0, 0)),
            pl.BlockSpec((1, D), lambda i: (0, 0)),
        ],
        out_specs=pl.BlockSpec((_BM, D), lambda i: (i, 0)),
        out_shape=jax.ShapeDtypeStruct((NP, D), jnp.float32),
    )(xpart, W, b2d)


@jax.jit
def kernel(poi_rep, edge_index, dist_vec, W, b):
    n1 = edge_index[0]
    n2 = edge_index[1]
    degp = _degree_kernel(n1)
    xpart = _edge_kernel(poi_rep, n1, n2, dist_vec, degp)
    msg = _matmul(xpart, W, jnp.reshape(b, (1, D)))
    return msg[:N]


# speedup vs baseline: 21.1184x; 1.8106x over previous
import jax
import jax.numpy as jnp
from jax import lax
from jax.experimental import pallas as pl
from jax.experimental.pallas import tpu as pltpu
from jax.experimental.pallas import tpu_sc as plsc

N = 10000
E = 320000
D = 128

NC = 2
NS = 16
L = 16
NW = NC * NS

NP = 10240
EW = E // NW
K = 80
NCHUNK = EW // K
ROWS_PER_TILE = NP // NS

_MESH = plsc.VectorSubcoreMesh(
    core_axis_name="c", subcore_axis_name="s", num_cores=NC, num_subcores=NS
)
_SC_PARAMS = pltpu.CompilerParams(needs_layout_passes=False)


def _rsqrt16(d):
    i = lax.bitcast_convert_type(d, jnp.int32)
    i = jnp.int32(0x5F3759DF) - lax.shift_right_logical(i, 1)
    r = lax.bitcast_convert_type(i, jnp.float32)
    half = d * jnp.float32(0.5)
    for _ in range(3):
        r = r * (jnp.float32(1.5) - half * r * r)
    return jnp.where(d == jnp.float32(0.0), jnp.float32(jnp.inf), r)


_DGRP = 8


def _degree_body(n1r_hbm, degp_hbm, idx2d_v, ones_v, zero_v, shared_deg, sem):
    c = lax.axis_index("c")
    s = lax.axis_index("s")
    wid = s * NC + c

    for k in range(K // L):
        ones_v[pl.ds(k * L, L)] = jnp.full((L,), 1.0, jnp.float32)
        zero_v[pl.ds(k * L, L)] = jnp.full((L,), 0.0, jnp.float32)

    for k in range(ROWS_PER_TILE // K):
        pltpu.sync_copy(zero_v, shared_deg.at[pl.ds(s * ROWS_PER_TILE + k * K, K)])

    pltpu.sync_copy(n1r_hbm.at[wid], idx2d_v)
    plsc.subcore_barrier()

    def group(g, carry):
        for j in range(_DGRP):
            pltpu.async_copy(ones_v, shared_deg.at[idx2d_v.at[g * _DGRP + j]],
                             sem, add=True)
        for j in range(_DGRP):
            pltpu.make_async_copy(ones_v, shared_deg.at[idx2d_v.at[g * _DGRP + j]],
                                  sem).wait()
        return carry

    lax.fori_loop(0, NCHUNK // _DGRP, group, 0)
    for i in range(NCHUNK - NCHUNK % _DGRP, NCHUNK):
        pltpu.async_copy(ones_v, shared_deg.at[idx2d_v.at[i]], sem, add=True)
    for i in range(NCHUNK - NCHUNK % _DGRP, NCHUNK):
        pltpu.make_async_copy(ones_v, shared_deg.at[idx2d_v.at[i]], sem).wait()
    plsc.subcore_barrier()

    pltpu.sync_copy(
        shared_deg.at[pl.ds(s * ROWS_PER_TILE, ROWS_PER_TILE)],
        degp_hbm.at[c, pl.ds(s * ROWS_PER_TILE, ROWS_PER_TILE)],
    )


_degree_kernel = pl.kernel(
    _degree_body,
    out_type=jax.ShapeDtypeStruct((NC, NP), jnp.float32),
    mesh=_MESH,
    compiler_params=_SC_PARAMS,
    scratch_types=[
        pltpu.VMEM((NCHUNK, K), jnp.int32),
        pltpu.VMEM((K,), jnp.float32),
        pltpu.VMEM((K,), jnp.float32),
        pltpu.VMEM_SHARED((NP,), jnp.float32),
        pltpu.SemaphoreType.DMA,
    ],
)


G = 5
NGRP = NCHUNK // G


GK = G * K


def _edge_body(
    poi_hbm, n1r_hbm, n2r_hbm, n1f_hbm, n2f_hbm, distf_hbm, degp_hbm,
    xpart_hbm,
    invdeg_v, p2_v, n1g_v, n2g_v, n1v_v, n2v_v, dv_v, rows_v, shared_acc,
    semg, sems, seme,
):
    c = lax.axis_index("c")
    s = lax.axis_index("s")
    wid = s * NC + c

    pltpu.sync_copy(degp_hbm.at[0], invdeg_v)
    NB = 640

    def invdeg_blk(t, carry):
        pltpu.sync_copy(degp_hbm.at[1, pl.ds(t * NB, NB)], p2_v)
        for k in range(NB // L):
            d = invdeg_v[pl.ds(t * NB + k * L, L)] + p2_v[pl.ds(k * L, L)]
            invdeg_v[pl.ds(t * NB + k * L, L)] = _rsqrt16(d)
        return carry

    lax.fori_loop(0, NP // NB, invdeg_blk, 0)

    for e in range(K):
        for j in range(D // L):
            rows_v[0, e, pl.ds(j * L, L)] = jnp.full((L,), 0.0, jnp.float32)
    for k in range(ROWS_PER_TILE // K):
        pltpu.sync_copy(rows_v.at[0],
                        shared_acc.at[pl.ds(s * ROWS_PER_TILE + k * K, K)])
    plsc.subcore_barrier()

    def load_group(g, p, sync):
        fbase = wid * EW + g * GK
        if sync:
            pltpu.sync_copy(n1r_hbm.at[wid, g], n1g_v.at[p])
            pltpu.sync_copy(n2r_hbm.at[wid, g], n2g_v.at[p])
            pltpu.sync_copy(n1f_hbm.at[pl.ds(fbase, GK)], n1v_v.at[pl.ds(p * GK, GK)])
            pltpu.sync_copy(n2f_hbm.at[pl.ds(fbase, GK)], n2v_v.at[pl.ds(p * GK, GK)])
            pltpu.sync_copy(distf_hbm.at[pl.ds(fbase, GK)], dv_v.at[pl.ds(p * GK, GK)])
        else:
            pltpu.async_copy(n1r_hbm.at[wid, g], n1g_v.at[p], seme)
            pltpu.async_copy(n2r_hbm.at[wid, g], n2g_v.at[p], seme)
            pltpu.async_copy(n1f_hbm.at[pl.ds(fbase, GK)],
                             n1v_v.at[pl.ds(p * GK, GK)], seme)
            pltpu.async_copy(n2f_hbm.at[pl.ds(fbase, GK)],
                             n2v_v.at[pl.ds(p * GK, GK)], seme)
            pltpu.async_copy(distf_hbm.at[pl.ds(fbase, GK)],
                             dv_v.at[pl.ds(p * GK, GK)], seme)

    def wait_group():
        pltpu.make_async_copy(n1r_hbm.at[0, 0], n1g_v.at[0], seme).wait()
        pltpu.make_async_copy(n2r_hbm.at[0, 0], n2g_v.at[0], seme).wait()
        pltpu.make_async_copy(n1f_hbm.at[pl.ds(0, GK)],
                              n1v_v.at[pl.ds(0, GK)], seme).wait()
        pltpu.make_async_copy(n2f_hbm.at[pl.ds(0, GK)],
                              n2v_v.at[pl.ds(0, GK)], seme).wait()
        pltpu.make_async_copy(distf_hbm.at[pl.ds(0, GK)],
                              dv_v.at[pl.ds(0, GK)], seme).wait()

    def scale(p, j, b):
        for k in range(K // L):
            base = p * GK + j * K + k * L
            n1v = n1v_v[pl.ds(base, L)]
            n2v = n2v_v[pl.ds(base, L)]
            w1 = plsc.load_gather(invdeg_v, [n1v])
            w2 = plsc.load_gather(invdeg_v, [n2v])
            d = dv_v[pl.ds(base, L)]
            wv = w1 * w2 * jnp.exp(-d * d)
            for l in range(L):
                e = k * L + l
                ws = jnp.full((L,), wv[l], jnp.float32)
                for jd in range(D // L):
                    rows_v[b, e, pl.ds(jd * L, L)] = (
                        rows_v[b, e, pl.ds(jd * L, L)] * ws)

    load_group(0, 0, True)
    load_group(1, 1, False)
    pltpu.async_copy(poi_hbm.at[n2g_v.at[0, 0]], rows_v.at[0], semg)

    def chunk(i, carry):
        g = i // G
        j = i % G
        b = i % 2
        p = g % 2

        pltpu.make_async_copy(poi_hbm.at[n2g_v.at[0, 0]], rows_v.at[b],
                              semg).wait()
        scale(p, j, b)

        @pl.when(i >= 1)
        def _():
            pltpu.make_async_copy(rows_v.at[1 - b],
                                  shared_acc.at[n1g_v.at[0, 0]], sems).wait()

        @pl.when(jnp.logical_and(j == 1,
                                 jnp.logical_and(g >= 1, g < NGRP - 1)))
        def _():
            load_group(g + 1, (g + 1) % 2, False)

        @pl.when(jnp.logical_and(j == G - 1, g < NGRP - 1))
        def _():
            wait_group()

        pltpu.async_copy(rows_v.at[b], shared_acc.at[n1g_v.at[p, j]], sems,
                         add=True)

        @pl.when(i + 1 < NCHUNK)
        def _():
            i1 = i + 1
            pltpu.async_copy(poi_hbm.at[n2g_v.at[(i1 // G) % 2, i1 % G]],
                             rows_v.at[1 - b], semg)
        return carry

    lax.fori_loop(0, NCHUNK, chunk, 0)
    pltpu.make_async_copy(rows_v.at[0], shared_acc.at[n1g_v.at[0, 0]],
                          sems).wait()
    plsc.subcore_barrier()

    pltpu.sync_copy(
        shared_acc.at[pl.ds(s * ROWS_PER_TILE, ROWS_PER_TILE)],
        xpart_hbm.at[c, pl.ds(s * ROWS_PER_TILE, ROWS_PER_TILE)],
    )


_edge_kernel = pl.kernel(
    _edge_body,
    out_type=jax.ShapeDtypeStruct((NC, NP, D), jnp.float32),
    mesh=_MESH,
    compiler_params=_SC_PARAMS,
    scratch_types=[
        pltpu.VMEM((NP,), jnp.float32),
        pltpu.VMEM((640,), jnp.float32),
        pltpu.VMEM((2, G, K), jnp.int32),
        pltpu.VMEM((2, G, K), jnp.int32),
        pltpu.VMEM((2 * GK,), jnp.int32),
        pltpu.VMEM((2 * GK,), jnp.int32),
        pltpu.VMEM((2 * GK,), jnp.float32),
        pltpu.VMEM((2, K, D), jnp.float32),
        pltpu.VMEM_SHARED((NP, D), jnp.float32),
        pltpu.SemaphoreType.DMA,
        pltpu.SemaphoreType.DMA,
        pltpu.SemaphoreType.DMA,
    ],
)


_BM = 1024


def _matmul_body(x_ref, w_ref, b_ref, o_ref):
    xs = x_ref[0] + x_ref[1]
    acc = lax.dot_general(
        xs, w_ref[...], (((1,), (1,)), ((), ())),
        preferred_element_type=jnp.float32,
    )
    o_ref[...] = acc + b_ref[...]


def _matmul(xpart, W, b2d):
    return pl.pallas_call(
        _matmul_body,
        grid=(NP // _BM,),
        in_specs=[
            pl.BlockSpec((NC, _BM, D), lambda i: (0, i, 0)),
            pl.BlockSpec((D, D), lambda i: (0, 0)),
            pl.BlockSpec((1, D), lambda i: (0, 0)),
        ],
        out_specs=pl.BlockSpec((_BM, D), lambda i: (i, 0)),
        out_shape=jax.ShapeDtypeStruct((NP, D), jnp.float32),
    )(xpart, W, b2d)


@jax.jit
def kernel(poi_rep, edge_index, dist_vec, W, b):
    n1 = edge_index[0]
    n2 = edge_index[1]
    n1r = jnp.reshape(n1, (NW, NGRP, G, K))
    n2r = jnp.reshape(n2, (NW, NGRP, G, K))
    degp = _degree_kernel(jnp.reshape(n1, (NW, NCHUNK, K)))
    xpart = _edge_kernel(poi_rep, n1r, n2r, n1, n2, dist_vec, degp)
    msg = _matmul(xpart, W, jnp.reshape(b, (1, D)))
    return msg[:N]


# speedup vs baseline: 25.7669x; 1.2201x over previous
import jax
import jax.numpy as jnp
from jax import lax
from jax.experimental import pallas as pl
from jax.experimental.pallas import tpu as pltpu
from jax.experimental.pallas import tpu_sc as plsc

N = 10000
E = 320000
D = 128

NC = 2
NS = 16
L = 16
NW = NC * NS

NP = 10240
EW = E // NW
K = 80
NCHUNK = EW // K
ROWS_PER_TILE = NP // NS

_MESH = plsc.VectorSubcoreMesh(
    core_axis_name="c", subcore_axis_name="s", num_cores=NC, num_subcores=NS
)
_SC_PARAMS = pltpu.CompilerParams(needs_layout_passes=False)


def _rsqrt16(d):
    i = lax.bitcast_convert_type(d, jnp.int32)
    i = jnp.int32(0x5F3759DF) - lax.shift_right_logical(i, 1)
    r = lax.bitcast_convert_type(i, jnp.float32)
    half = d * jnp.float32(0.5)
    for _ in range(3):
        r = r * (jnp.float32(1.5) - half * r * r)
    return jnp.where(d == jnp.float32(0.0), jnp.float32(jnp.inf), r)


_DGRP = 8


def _degree_body(n1r_hbm, degp_hbm, idx2d_v, ones_v, zero_v, shared_deg, sem):
    c = lax.axis_index("c")
    s = lax.axis_index("s")
    wid = s * NC + c

    for k in range(K // L):
        ones_v[pl.ds(k * L, L)] = jnp.full((L,), 1.0, jnp.float32)
        zero_v[pl.ds(k * L, L)] = jnp.full((L,), 0.0, jnp.float32)

    for k in range(ROWS_PER_TILE // K):
        pltpu.sync_copy(zero_v, shared_deg.at[pl.ds(s * ROWS_PER_TILE + k * K, K)])

    pltpu.sync_copy(n1r_hbm.at[wid], idx2d_v)
    plsc.subcore_barrier()

    def group(g, carry):
        for j in range(_DGRP):
            pltpu.async_copy(ones_v, shared_deg.at[idx2d_v.at[g * _DGRP + j]],
                             sem, add=True)
        for j in range(_DGRP):
            pltpu.make_async_copy(ones_v, shared_deg.at[idx2d_v.at[g * _DGRP + j]],
                                  sem).wait()
        return carry

    lax.fori_loop(0, NCHUNK // _DGRP, group, 0)
    for i in range(NCHUNK - NCHUNK % _DGRP, NCHUNK):
        pltpu.async_copy(ones_v, shared_deg.at[idx2d_v.at[i]], sem, add=True)
    for i in range(NCHUNK - NCHUNK % _DGRP, NCHUNK):
        pltpu.make_async_copy(ones_v, shared_deg.at[idx2d_v.at[i]], sem).wait()
    plsc.subcore_barrier()

    pltpu.sync_copy(
        shared_deg.at[pl.ds(s * ROWS_PER_TILE, ROWS_PER_TILE)],
        degp_hbm.at[c, pl.ds(s * ROWS_PER_TILE, ROWS_PER_TILE)],
    )


_degree_kernel = pl.kernel(
    _degree_body,
    out_type=jax.ShapeDtypeStruct((NC, NP), jnp.float32),
    mesh=_MESH,
    compiler_params=_SC_PARAMS,
    scratch_types=[
        pltpu.VMEM((NCHUNK, K), jnp.int32),
        pltpu.VMEM((K,), jnp.float32),
        pltpu.VMEM((K,), jnp.float32),
        pltpu.VMEM_SHARED((NP,), jnp.float32),
        pltpu.SemaphoreType.DMA,
    ],
)


G = 5
NGRP = NCHUNK // G


GK = G * K


def _edge_body(
    poi_hbm, n1r_hbm, n2r_hbm, n1f_hbm, n2f_hbm, distf_hbm, degp_hbm,
    xpart_hbm,
    invdeg_v, p2_v, n1g_v, n2g_v, n1v_v, n2v_v, dv_v, rows_v, shared_acc,
    semg, sems, seme,
):
    c = lax.axis_index("c")
    s = lax.axis_index("s")
    wid = s * NC + c

    pltpu.sync_copy(degp_hbm.at[0], invdeg_v)
    NB = 640

    def invdeg_blk(t, carry):
        pltpu.sync_copy(degp_hbm.at[1, pl.ds(t * NB, NB)], p2_v)
        for k in range(NB // L):
            d = invdeg_v[pl.ds(t * NB + k * L, L)] + p2_v[pl.ds(k * L, L)]
            invdeg_v[pl.ds(t * NB + k * L, L)] = _rsqrt16(d)
        return carry

    lax.fori_loop(0, NP // NB, invdeg_blk, 0)

    for e in range(K):
        for j in range(D // L):
            rows_v[0, e, pl.ds(j * L, L)] = jnp.full((L,), 0.0, jnp.float32)
    for k in range(ROWS_PER_TILE // K):
        pltpu.sync_copy(rows_v.at[0],
                        shared_acc.at[pl.ds(s * ROWS_PER_TILE + k * K, K)])
    plsc.subcore_barrier()

    def load_group(g, p, sync):
        fbase = wid * EW + g * GK
        if sync:
            pltpu.sync_copy(n1r_hbm.at[wid, g], n1g_v.at[p])
            pltpu.sync_copy(n2r_hbm.at[wid, g], n2g_v.at[p])
            pltpu.sync_copy(n1f_hbm.at[pl.ds(fbase, GK)], n1v_v.at[pl.ds(p * GK, GK)])
            pltpu.sync_copy(n2f_hbm.at[pl.ds(fbase, GK)], n2v_v.at[pl.ds(p * GK, GK)])
            pltpu.sync_copy(distf_hbm.at[pl.ds(fbase, GK)], dv_v.at[pl.ds(p * GK, GK)])
        else:
            pltpu.async_copy(n1r_hbm.at[wid, g], n1g_v.at[p], seme)
            pltpu.async_copy(n2r_hbm.at[wid, g], n2g_v.at[p], seme)
            pltpu.async_copy(n1f_hbm.at[pl.ds(fbase, GK)],
                             n1v_v.at[pl.ds(p * GK, GK)], seme)
            pltpu.async_copy(n2f_hbm.at[pl.ds(fbase, GK)],
                             n2v_v.at[pl.ds(p * GK, GK)], seme)
            pltpu.async_copy(distf_hbm.at[pl.ds(fbase, GK)],
                             dv_v.at[pl.ds(p * GK, GK)], seme)

    def wait_group():
        pltpu.make_async_copy(n1r_hbm.at[0, 0], n1g_v.at[0], seme).wait()
        pltpu.make_async_copy(n2r_hbm.at[0, 0], n2g_v.at[0], seme).wait()
        pltpu.make_async_copy(n1f_hbm.at[pl.ds(0, GK)],
                              n1v_v.at[pl.ds(0, GK)], seme).wait()
        pltpu.make_async_copy(n2f_hbm.at[pl.ds(0, GK)],
                              n2v_v.at[pl.ds(0, GK)], seme).wait()
        pltpu.make_async_copy(distf_hbm.at[pl.ds(0, GK)],
                              dv_v.at[pl.ds(0, GK)], seme).wait()

    def scale(p, j, b):
        for k in range(K // L):
            base = p * GK + j * K + k * L
            n1v = n1v_v[pl.ds(base, L)]
            n2v = n2v_v[pl.ds(base, L)]
            w1 = plsc.load_gather(invdeg_v, [n1v])
            w2 = plsc.load_gather(invdeg_v, [n2v])
            d = dv_v[pl.ds(base, L)]
            wv = w1 * w2 * jnp.exp(-d * d)
            for l in range(L):
                e = k * L + l
                ws = jnp.full((L,), wv[l], jnp.float32)
                for jd in range(D // L):
                    rows_v[b, e, pl.ds(jd * L, L)] = (
                        rows_v[b, e, pl.ds(jd * L, L)] * ws)

    load_group(0, 0, True)
    load_group(1, 1, False)
    pltpu.async_copy(poi_hbm.at[n2g_v.at[0, 0]], rows_v.at[0], semg)

    def chunk(i, carry):
        g = i // G
        j = i % G
        b = i % 2
        p = g % 2

        pltpu.make_async_copy(poi_hbm.at[n2g_v.at[0, 0]], rows_v.at[b],
                              semg).wait()

        @pl.when(i >= 1)
        def _():
            pltpu.make_async_copy(rows_v.at[1 - b],
                                  shared_acc.at[n1g_v.at[0, 0]], sems).wait()

        @pl.when(jnp.logical_and(j == G - 1, g < NGRP - 1))
        def _():
            wait_group()

        @pl.when(i + 1 < NCHUNK)
        def _():
            i1 = i + 1
            pltpu.async_copy(poi_hbm.at[n2g_v.at[(i1 // G) % 2, i1 % G]],
                             rows_v.at[1 - b], semg)

        @pl.when(jnp.logical_and(j == 1,
                                 jnp.logical_and(g >= 1, g < NGRP - 1)))
        def _():
            load_group(g + 1, (g + 1) % 2, False)

        scale(p, j, b)

        pltpu.async_copy(rows_v.at[b], shared_acc.at[n1g_v.at[p, j]], sems,
                         add=True)
        return carry

    lax.fori_loop(0, NCHUNK, chunk, 0)
    pltpu.make_async_copy(rows_v.at[0], shared_acc.at[n1g_v.at[0, 0]],
                          sems).wait()
    plsc.subcore_barrier()

    pltpu.sync_copy(
        shared_acc.at[pl.ds(s * ROWS_PER_TILE, ROWS_PER_TILE)],
        xpart_hbm.at[c, pl.ds(s * ROWS_PER_TILE, ROWS_PER_TILE)],
    )


_edge_kernel = pl.kernel(
    _edge_body,
    out_type=jax.ShapeDtypeStruct((NC, NP, D), jnp.float32),
    mesh=_MESH,
    compiler_params=_SC_PARAMS,
    scratch_types=[
        pltpu.VMEM((NP,), jnp.float32),
        pltpu.VMEM((640,), jnp.float32),
        pltpu.VMEM((2, G, K), jnp.int32),
        pltpu.VMEM((2, G, K), jnp.int32),
        pltpu.VMEM((2 * GK,), jnp.int32),
        pltpu.VMEM((2 * GK,), jnp.int32),
        pltpu.VMEM((2 * GK,), jnp.float32),
        pltpu.VMEM((2, K, D), jnp.float32),
        pltpu.VMEM_SHARED((NP, D), jnp.float32),
        pltpu.SemaphoreType.DMA,
        pltpu.SemaphoreType.DMA,
        pltpu.SemaphoreType.DMA,
    ],
)


_BM = 1024


def _matmul_body(x_ref, w_ref, b_ref, o_ref):
    xs = x_ref[0] + x_ref[1]
    acc = lax.dot_general(
        xs, w_ref[...], (((1,), (1,)), ((), ())),
        preferred_element_type=jnp.float32,
    )
    o_ref[...] = acc + b_ref[...]


def _matmul(xpart, W, b2d):
    return pl.pallas_call(
        _matmul_body,
        grid=(NP // _BM,),
        in_specs=[
            pl.BlockSpec((NC, _BM, D), lambda i: (0, i, 0)),
            pl.BlockSpec((D, D), lambda i: (0, 0)),
            pl.BlockSpec((1, D), lambda i: (0, 0)),
        ],
        out_specs=pl.BlockSpec((_BM, D), lambda i: (i, 0)),
        out_shape=jax.ShapeDtypeStruct((NP, D), jnp.float32),
    )(xpart, W, b2d)


@jax.jit
def kernel(poi_rep, edge_index, dist_vec, W, b):
    n1 = edge_index[0]
    n2 = edge_index[1]
    n1r = jnp.reshape(n1, (NW, NGRP, G, K))
    n2r = jnp.reshape(n2, (NW, NGRP, G, K))
    degp = _degree_kernel(jnp.reshape(n1, (NW, NCHUNK, K)))
    xpart = _edge_kernel(poi_rep, n1r, n2r, n1, n2, dist_vec, degp)
    msg = _matmul(xpart, W, jnp.reshape(b, (1, D)))
    return msg[:N]
